# baseline (speedup 1.0000x reference)
import functools

import jax
import jax.numpy as jnp
from jax import lax
from jax.experimental import pallas as pl
from jax.experimental.pallas import tpu as pltpu
from jax.experimental.pallas import tpu_sc as plsc

NUM_VARS = 512
HIDDEN = 16
TOPK = 8
TEMP = 1.0

_NC = 2
_NS = 16
_LANES = 16
_ROWS_PER_W = NUM_VARS // (_NC * _NS)


def _sim_kernel(ve_ref, wq_ref, bq_ref, wk_ref, bk_ref, sim_ref):
    ve = ve_ref[...]
    q = lax.dot_general(ve, wq_ref[...], (((1,), (1,)), ((), ())),
                        preferred_element_type=jnp.float32) + bq_ref[...]
    k = lax.dot_general(ve, wk_ref[...], (((1,), (1,)), ((), ())),
                        preferred_element_type=jnp.float32) + bk_ref[...]
    sim = lax.dot_general(q, k, (((1,), (1,)), ((), ())),
                          preferred_element_type=jnp.float32)
    n = sim.shape[0]
    row = lax.broadcasted_iota(jnp.int32, (n, n), 0)
    col = lax.broadcasted_iota(jnp.int32, (n, n), 1)
    sim_ref[...] = jnp.where(row == col, jnp.float32(-1e9), sim)


def _sc_topk_body(sim_hbm, w_hbm, idx_hbm, sim_tile, w_tile, idx_tile):
    wid = lax.axis_index("s") * _NC + lax.axis_index("c")
    base = wid * _ROWS_PER_W
    pltpu.sync_copy(sim_hbm.at[pl.ds(base, _ROWS_PER_W)], sim_tile)

    lane = lax.iota(jnp.int32, _LANES)
    mask8 = lane < TOPK

    def one_row(r):
        nodes = []
        for j in range(NUM_VARS // _LANES):
            kj = sim_tile[r, pl.ds(j * _LANES, _LANES)]
            vj = lane + (j * _LANES)
            nodes.append(plsc.sort_key_val(kj, vj, descending=True))
        while len(nodes) > 1:
            nxt = []
            for i in range(0, len(nodes), 2):
                ak, av = nodes[i]
                bk, bv = nodes[i + 1]
                mk = jnp.where(mask8, ak, lax.rev(bk, (0,)))
                mv = jnp.where(mask8, av, lax.rev(bv, (0,)))
                nxt.append(plsc.sort_key_val(mk, mv, descending=True))
            nodes = nxt
        kf, vf = nodes[0]

        m0 = jnp.max(kf)
        e = jnp.where(mask8, jnp.exp((kf - m0) * jnp.float32(1.0 / TEMP)),
                      jnp.float32(0.0))
        denom = jnp.broadcast_to(jnp.sum(e), (_LANES,))
        w_tile[r, :] = e / denom
        idx_tile[r, :] = vf

    def row_body(r2, _):
        one_row(r2 * 2)
        one_row(r2 * 2 + 1)
        return ()

    lax.fori_loop(0, _ROWS_PER_W // 2, row_body, ())
    pltpu.sync_copy(w_tile, w_hbm.at[pl.ds(base, _ROWS_PER_W)])
    pltpu.sync_copy(idx_tile, idx_hbm.at[pl.ds(base, _ROWS_PER_W)])


_sc_topk = functools.partial(
    pl.kernel,
    out_type=(jax.ShapeDtypeStruct((NUM_VARS, _LANES), jnp.float32),
              jax.ShapeDtypeStruct((NUM_VARS, _LANES), jnp.int32)),
    mesh=plsc.VectorSubcoreMesh(core_axis_name="c", subcore_axis_name="s"),
    compiler_params=pltpu.CompilerParams(needs_layout_passes=False),
    scratch_types=[
        pltpu.VMEM((_ROWS_PER_W, NUM_VARS), jnp.float32),
        pltpu.VMEM((_ROWS_PER_W, _LANES), jnp.float32),
        pltpu.VMEM((_ROWS_PER_W, _LANES), jnp.int32),
    ],
)(_sc_topk_body)


def _combine_kernel(w_ref, idx_ref, x_ref, o_ref, s_scr):
    @pl.when(pl.program_id(0) == 0)
    def _():
        n = NUM_VARS
        col = lax.broadcasted_iota(jnp.int32, (n, n), 1)
        s = jnp.zeros((n, n), jnp.float32)
        for k in range(TOPK):
            idxk = idx_ref[:, k:k + 1]
            wk = w_ref[:, k:k + 1]
            s = jnp.where(col == idxk, wk, s)
        s_scr[...] = s

    o_ref[...] = lax.dot_general(
        x_ref[...], s_scr[...], (((1,), (1,)), ((), ())),
        preferred_element_type=jnp.float32)


@jax.jit
def kernel(x, var_embed, Wq, bq, Wk, bk):
    Bsz, L, N = x.shape
    ve = var_embed.reshape(N, HIDDEN)

    sim = pl.pallas_call(
        _sim_kernel,
        out_shape=jax.ShapeDtypeStruct((N, N), jnp.float32),
    )(ve, Wq, bq.reshape(1, HIDDEN), Wk, bk.reshape(1, HIDDEN))

    w, idx = _sc_topk(sim)

    xs = x.reshape(Bsz * L, N)
    BL = 4096
    grid = (Bsz * L) // BL
    out = pl.pallas_call(
        _combine_kernel,
        grid=(grid,),
        in_specs=[
            pl.BlockSpec((N, _LANES), lambda i: (0, 0)),
            pl.BlockSpec((N, _LANES), lambda i: (0, 0)),
            pl.BlockSpec((BL, N), lambda i: (i, 0)),
        ],
        out_specs=pl.BlockSpec((BL, N), lambda i: (i, 0)),
        out_shape=jax.ShapeDtypeStruct((Bsz * L, N), jnp.float32),
        scratch_shapes=[pltpu.VMEM((N, N), jnp.float32)],
    )(w, idx, xs)
    return out.reshape(Bsz, L, N)


# speedup vs baseline: 1.0339x; 1.0339x over previous
import functools

import jax
import jax.numpy as jnp
from jax import lax
from jax.experimental import pallas as pl
from jax.experimental.pallas import tpu as pltpu
from jax.experimental.pallas import tpu_sc as plsc

NUM_VARS = 512
HIDDEN = 16
TOPK = 8
TEMP = 1.0

_NC = 2
_NS = 16
_LANES = 16
_ROWS_PER_W = NUM_VARS // (_NC * _NS)


def _sim_kernel(ve_ref, wq_ref, bq_ref, wk_ref, bk_ref, sim_ref):
    ve = ve_ref[...]
    q = lax.dot_general(ve, wq_ref[...], (((1,), (1,)), ((), ())),
                        preferred_element_type=jnp.float32) + bq_ref[...]
    k = lax.dot_general(ve, wk_ref[...], (((1,), (1,)), ((), ())),
                        preferred_element_type=jnp.float32) + bk_ref[...]
    sim = lax.dot_general(q, k, (((1,), (1,)), ((), ())),
                          preferred_element_type=jnp.float32)
    n = sim.shape[0]
    row = lax.broadcasted_iota(jnp.int32, (n, n), 0)
    col = lax.broadcasted_iota(jnp.int32, (n, n), 1)
    sim_ref[...] = jnp.where(row == col, jnp.float32(-1e9), sim)


def _sc_routing_body(sim_hbm, s_hbm, sim_tile, s_tile):
    wid = lax.axis_index("s") * _NC + lax.axis_index("c")
    base = wid * _ROWS_PER_W
    pltpu.sync_copy(sim_hbm.at[pl.ds(base, _ROWS_PER_W)], sim_tile)

    lane = lax.iota(jnp.int32, _LANES)
    mask8 = lane < TOPK

    def one_row(r):
        nodes = []
        for j in range(NUM_VARS // _LANES):
            kj = sim_tile[r, pl.ds(j * _LANES, _LANES)]
            vj = lane + (j * _LANES)
            nodes.append(plsc.sort_key_val(kj, vj, descending=True))
        while len(nodes) > 1:
            nxt = []
            for i in range(0, len(nodes), 2):
                ak, av = nodes[i]
                bk, bv = nodes[i + 1]
                mk = jnp.where(mask8, ak, lax.rev(bk, (0,)))
                mv = jnp.where(mask8, av, lax.rev(bv, (0,)))
                nxt.append(plsc.sort_key_val(mk, mv, descending=True))
            nodes = nxt
        kf, vf = nodes[0]

        m0 = jnp.max(kf)
        e = jnp.where(mask8, jnp.exp((kf - m0) * jnp.float32(1.0 / TEMP)),
                      jnp.float32(0.0))
        denom = jnp.broadcast_to(jnp.sum(e), (_LANES,))
        w = e / denom

        zero = jnp.zeros((_LANES,), jnp.float32)
        for j in range(NUM_VARS // _LANES):
            s_tile[r, pl.ds(j * _LANES, _LANES)] = zero
        plsc.store_scatter(s_tile, [jnp.full((_LANES,), r, jnp.int32), vf],
                           w, mask=mask8)

    def row_body(r2, _):
        one_row(r2 * 2)
        one_row(r2 * 2 + 1)
        return ()

    lax.fori_loop(0, _ROWS_PER_W // 2, row_body, ())
    pltpu.sync_copy(s_tile, s_hbm.at[pl.ds(base, _ROWS_PER_W)])


_sc_routing = functools.partial(
    pl.kernel,
    out_type=jax.ShapeDtypeStruct((NUM_VARS, NUM_VARS), jnp.float32),
    mesh=plsc.VectorSubcoreMesh(core_axis_name="c", subcore_axis_name="s"),
    compiler_params=pltpu.CompilerParams(needs_layout_passes=False),
    scratch_types=[
        pltpu.VMEM((_ROWS_PER_W, NUM_VARS), jnp.float32),
        pltpu.VMEM((_ROWS_PER_W, NUM_VARS), jnp.float32),
    ],
)(_sc_routing_body)


def _combine_kernel(x_ref, s_ref, o_ref):
    o_ref[...] = lax.dot_general(
        x_ref[...], s_ref[...], (((1,), (1,)), ((), ())),
        preferred_element_type=jnp.float32)


@jax.jit
def kernel(x, var_embed, Wq, bq, Wk, bk):
    Bsz, L, N = x.shape
    ve = var_embed.reshape(N, HIDDEN)

    sim = pl.pallas_call(
        _sim_kernel,
        out_shape=jax.ShapeDtypeStruct((N, N), jnp.float32),
    )(ve, Wq, bq.reshape(1, HIDDEN), Wk, bk.reshape(1, HIDDEN))

    s = _sc_routing(sim)

    xs = x.reshape(Bsz * L, N)
    BL = 4096
    grid = (Bsz * L) // BL
    out = pl.pallas_call(
        _combine_kernel,
        grid=(grid,),
        in_specs=[
            pl.BlockSpec((BL, N), lambda i: (i, 0)),
            pl.BlockSpec((N, N), lambda i: (0, 0)),
        ],
        out_specs=pl.BlockSpec((BL, N), lambda i: (i, 0)),
        out_shape=jax.ShapeDtypeStruct((Bsz * L, N), jnp.float32),
    )(xs, s)
    return out.reshape(Bsz, L, N)


# speedup vs baseline: 2.1875x; 2.1158x over previous
import functools

import jax
import jax.numpy as jnp
from jax import lax
from jax.experimental import pallas as pl
from jax.experimental.pallas import tpu as pltpu

NUM_VARS = 512
HIDDEN = 16
TOPK = 8
TEMP = 1.0


def _compute_s(ve_ref, wq_ref, bq_ref, wk_ref, bk_ref):
    ve = ve_ref[...]
    q = lax.dot_general(ve, wq_ref[...], (((1,), (1,)), ((), ())),
                        preferred_element_type=jnp.float32) + bq_ref[...]
    k = lax.dot_general(ve, wk_ref[...], (((1,), (1,)), ((), ())),
                        preferred_element_type=jnp.float32) + bk_ref[...]
    sim = lax.dot_general(q, k, (((1,), (1,)), ((), ())),
                          preferred_element_type=jnp.float32)
    n = sim.shape[0]
    row = lax.broadcasted_iota(jnp.int32, (n, n), 0)
    col = lax.broadcasted_iota(jnp.int32, (n, n), 1)
    sim = jnp.where(row == col, jnp.float32(-1e9), sim)

    cur = sim
    s_acc = jnp.zeros_like(sim)
    denom = jnp.zeros((n, 1), jnp.float32)
    m0 = None
    for step in range(TOPK):
        m = jnp.max(cur, axis=1, keepdims=True)
        if step == 0:
            m0 = m
        first_col = jnp.min(jnp.where(cur == m, col, n), axis=1, keepdims=True)
        sel = col == first_col
        w = jnp.exp((m - m0) * jnp.float32(1.0 / TEMP))
        s_acc = jnp.where(sel, w, s_acc)
        denom = denom + w
        cur = jnp.where(sel, jnp.float32(-3e38), cur)
    return s_acc / denom


def _fused_kernel(ve_ref, wq_ref, bq_ref, wk_ref, bk_ref, x_ref, o_ref, s_scr):
    @pl.when(pl.program_id(0) == 0)
    def _():
        s_scr[...] = _compute_s(ve_ref, wq_ref, bq_ref, wk_ref, bk_ref)

    o_ref[...] = lax.dot_general(
        x_ref[...], s_scr[...], (((1,), (1,)), ((), ())),
        preferred_element_type=jnp.float32)


@jax.jit
def kernel(x, var_embed, Wq, bq, Wk, bk):
    Bsz, L, N = x.shape
    ve = var_embed.reshape(N, HIDDEN)

    xs = x.reshape(Bsz * L, N)
    BL = 4096
    grid = (Bsz * L) // BL
    out = pl.pallas_call(
        _fused_kernel,
        grid=(grid,),
        in_specs=[
            pl.BlockSpec((N, HIDDEN), lambda i: (0, 0)),
            pl.BlockSpec((HIDDEN, HIDDEN), lambda i: (0, 0)),
            pl.BlockSpec((1, HIDDEN), lambda i: (0, 0)),
            pl.BlockSpec((HIDDEN, HIDDEN), lambda i: (0, 0)),
            pl.BlockSpec((1, HIDDEN), lambda i: (0, 0)),
            pl.BlockSpec((BL, N), lambda i: (i, 0)),
        ],
        out_specs=pl.BlockSpec((BL, N), lambda i: (i, 0)),
        out_shape=jax.ShapeDtypeStruct((Bsz * L, N), jnp.float32),
        scratch_shapes=[pltpu.VMEM((N, N), jnp.float32)],
    )(ve, Wq, bq.reshape(1, HIDDEN), Wk, bk.reshape(1, HIDDEN), xs)
    return out.reshape(Bsz, L, N)


# speedup vs baseline: 2.1901x; 1.0012x over previous
import functools

import jax
import jax.numpy as jnp
from jax import lax
from jax.experimental import pallas as pl
from jax.experimental.pallas import tpu as pltpu

NUM_VARS = 512
HIDDEN = 16
TOPK = 8
TEMP = 1.0


def _compute_s(ve_ref, wq_ref, bq_ref, wk_ref, bk_ref):
    ve = ve_ref[0]
    q = lax.dot_general(ve, wq_ref[...], (((1,), (1,)), ((), ())),
                        preferred_element_type=jnp.float32) + bq_ref[...]
    k = lax.dot_general(ve, wk_ref[...], (((1,), (1,)), ((), ())),
                        preferred_element_type=jnp.float32) + bk_ref[...]
    sim = lax.dot_general(q, k, (((1,), (1,)), ((), ())),
                          preferred_element_type=jnp.float32)
    n = sim.shape[0]
    row = lax.broadcasted_iota(jnp.int32, (n, n), 0)
    col = lax.broadcasted_iota(jnp.int32, (n, n), 1)
    sim = jnp.where(row == col, jnp.float32(-1e9), sim)

    cur = sim
    s_acc = jnp.zeros_like(sim)
    denom = jnp.zeros((n, 1), jnp.float32)
    m0 = None
    for step in range(TOPK):
        m = jnp.max(cur, axis=1, keepdims=True)
        if step == 0:
            m0 = m
        first_col = jnp.min(jnp.where(cur == m, col, n), axis=1, keepdims=True)
        sel = col == first_col
        w = jnp.exp((m - m0) * jnp.float32(1.0 / TEMP))
        s_acc = jnp.where(sel, w, s_acc)
        denom = denom + w
        cur = jnp.where(sel, jnp.float32(-3e38), cur)
    return s_acc / denom


def _fused_kernel(ve_ref, wq_ref, bq_ref, wk_ref, bk_ref, x_ref, o_ref, s_scr):
    first = (pl.program_id(0) == 0) & (pl.program_id(1) == 0)

    @pl.when(first)
    def _():
        s_scr[...] = _compute_s(ve_ref, wq_ref, bq_ref, wk_ref, bk_ref)

    o_ref[0] = lax.dot_general(
        x_ref[0], s_scr[...], (((1,), (1,)), ((), ())),
        preferred_element_type=jnp.float32)


@jax.jit
def kernel(x, var_embed, Wq, bq, Wk, bk):
    Bsz, L, N = x.shape
    BL = 4096
    out = pl.pallas_call(
        _fused_kernel,
        grid=(Bsz, L // BL),
        in_specs=[
            pl.BlockSpec((1, N, HIDDEN), lambda b, i: (0, 0, 0)),
            pl.BlockSpec((HIDDEN, HIDDEN), lambda b, i: (0, 0)),
            pl.BlockSpec((1, HIDDEN), lambda b, i: (0, 0)),
            pl.BlockSpec((HIDDEN, HIDDEN), lambda b, i: (0, 0)),
            pl.BlockSpec((1, HIDDEN), lambda b, i: (0, 0)),
            pl.BlockSpec((1, BL, N), lambda b, i: (b, i, 0)),
        ],
        out_specs=pl.BlockSpec((1, BL, N), lambda b, i: (b, i, 0)),
        out_shape=jax.ShapeDtypeStruct((Bsz, L, N), jnp.float32),
        scratch_shapes=[pltpu.VMEM((N, N), jnp.float32)],
    )(var_embed, Wq, bq.reshape(1, HIDDEN), Wk, bk.reshape(1, HIDDEN), x)
    return out
